# baseline (speedup 1.0000x reference)
import functools

import jax
import jax.numpy as jnp
from jax import lax
from jax.experimental import pallas as pl
from jax.experimental.pallas import tpu as pltpu
from jax.experimental.pallas import tpu_sc as plsc

N = 10000
D = 128
DH = D // 2
K = 10
E = 320000

NC, NS, L = 2, 16, 16
NW = NC * NS
CH = 128
NCH = 160
NPH = 5
CPP = NCH // NPH
EPT = CH * NCH
EPAD = EPT * NS
NP2 = N // 2
NPAD2 = 5120
NPAD = 2 * NPAD2
RPS = NPAD2 // NS
BM = 1000

_sc_mesh = plsc.VectorSubcoreMesh(
    core_axis_name="c", subcore_axis_name="s", num_cores=NC, num_subcores=NS)
_sc_params = pltpu.CompilerParams(needs_layout_passes=False)


@functools.partial(
    pl.kernel,
    out_type=jax.ShapeDtypeStruct((NW, NPAD), jnp.float32),
    mesh=_sc_mesh,
    compiler_params=_sc_params,
    scratch_types=[
        pltpu.VMEM((EPAD // NW,), jnp.int32),
        pltpu.VMEM((EPAD // NW,), jnp.float32),
        pltpu.VMEM((NPAD,), jnp.float32),
    ],
)
def _sc_degree(col_hbm, ew_hbm, out_hbm, col_v, ew_v, deg_v):
    cid = lax.axis_index("c")
    sid = lax.axis_index("s")
    wid = sid * NC + cid
    pltpu.sync_copy(col_hbm.at[wid], col_v)
    pltpu.sync_copy(ew_hbm.at[wid], ew_v)
    zeros = jnp.zeros((L,), jnp.float32)

    def zero_body(k, _):
        deg_v[pl.ds(k * L, L)] = zeros
        return 0

    lax.fori_loop(0, NPAD // L, zero_body, 0)

    def edge_body(k, _):
        idx = col_v[pl.ds(k * L, L)]
        val = ew_v[pl.ds(k * L, L)]
        plsc.addupdate_scatter(deg_v, [idx], val)
        return 0

    lax.fori_loop(0, (EPAD // NW) // L, edge_body, 0)
    pltpu.sync_copy(deg_v, out_hbm.at[wid])


@functools.partial(
    pl.kernel,
    out_type=jax.ShapeDtypeStruct((NC, NPAD2, D), jnp.float32),
    mesh=_sc_mesh,
    compiler_params=_sc_params,
    scratch_types=[
        pltpu.VMEM((CPP, CH), jnp.int32),
        pltpu.VMEM((CPP, CH), jnp.int32),
        pltpu.VMEM((CPP, CH), jnp.int32),
        pltpu.VMEM((CPP, CH), jnp.float32),
        pltpu.VMEM((CH, D), jnp.float32),
        pltpu.VMEM((CH, D), jnp.float32),
        pltpu.VMEM_SHARED((NP2, D), jnp.float32),
        pltpu.VMEM_SHARED((NPAD2, D), jnp.float32),
        pltpu.SemaphoreType.DMA,
        pltpu.SemaphoreType.DMA,
    ],
)
def _sc_edge_pass(g_hbm, rp_hbm, cp_hbm, par_hbm, ew_hbm, out_hbm,
                  rp_v, cp_v, par_v, ew_v, buf0, buf1, g_sh, acc_sh,
                  sem0, sem1):
    cid = lax.axis_index("c")
    sid = lax.axis_index("s")

    @pl.when(sid < NS - 1)
    def _():
        off = pl.multiple_of(sid * 312, 8)
        pltpu.sync_copy(g_hbm.at[cid, pl.ds(off, 312)],
                        g_sh.at[pl.ds(off, 312)])

    @pl.when(sid == NS - 1)
    def _():
        pltpu.sync_copy(g_hbm.at[cid, pl.ds(4680, 320)],
                        g_sh.at[pl.ds(4680, 320)])

    zeros = jnp.zeros((L,), jnp.float32)

    def zrow(i, _):
        for v in range(D // L):
            buf0[i, pl.ds(v * L, L)] = zeros
        return 0

    lax.fori_loop(0, CH, zrow, 0)
    base = sid * RPS
    for j in range(RPS // 64):
        pltpu.sync_copy(buf0.at[pl.ds(0, 64)],
                        acc_sh.at[pl.ds(base + j * 64, 64)])
    plsc.subcore_barrier()

    bufs = (buf0, buf1)
    sems = (sem0, sem1)

    def scale(buf, j):
        def edge(b, _):
            w16 = ew_v[j, pl.ds(b * L, L)]
            p16 = par_v[j, pl.ds(b * L, L)]
            for lane in range(L):
                i = b * L + lane
                w = w16[lane]
                pv = p16[lane]
                pr = (pv & 1) == 0
                pc = (pv >> 1) == 0
                t = [buf[i, pl.ds(v * L, L)] for v in range(D // L)]
                for v in range(DH // L):
                    src = jnp.where(pr, t[v], t[v + DH // L]) * w
                    buf[i, pl.ds(v * L, L)] = jnp.where(pc, src, zeros)
                    buf[i, pl.ds((v + DH // L) * L, L)] = \
                        jnp.where(pc, zeros, src)
            return 0

        lax.fori_loop(0, CH // L, edge, 0)

    for ph in range(NPH):
        pltpu.sync_copy(rp_hbm.at[sid, pl.ds(ph * CPP, CPP)], rp_v)
        pltpu.sync_copy(cp_hbm.at[sid, pl.ds(ph * CPP, CPP)], cp_v)
        pltpu.sync_copy(par_hbm.at[sid, pl.ds(ph * CPP, CPP)], par_v)
        pltpu.sync_copy(ew_hbm.at[sid, pl.ds(ph * CPP, CPP)], ew_v)
        pltpu.async_copy(g_sh.at[rp_v.at[0]], bufs[0], sems[0])

        def pair(p, _):
            for q in range(2):
                j = 2 * p + q
                buf, nbuf = bufs[q], bufs[1 - q]
                sem, nsem = sems[q], sems[1 - q]
                pltpu.make_async_copy(g_sh.at[rp_v.at[j]], buf, sem).wait()

                @pl.when(j + 1 < CPP)
                def _():
                    pltpu.async_copy(g_sh.at[rp_v.at[j + 1]], nbuf, nsem)

                scale(buf, j)
                pltpu.sync_copy(buf, acc_sh.at[cp_v.at[j]], add=True)
            return 0

        lax.fori_loop(0, CPP // 2, pair, 0)

    plsc.subcore_barrier()
    for j in range(RPS // 64):
        sl = pl.ds(base + j * 64, 64)
        pltpu.sync_copy(acc_sh.at[sl], out_hbm.at[cid, sl])


def _dis_body(degp_ref, dis_ref):
    deg = jnp.sum(degp_ref[...], axis=0) + 1.0
    dis_ref[...] = jnp.where(deg > 0, lax.rsqrt(deg), 0.0)


_dis_call = pl.pallas_call(
    _dis_body,
    out_shape=jax.ShapeDtypeStruct((NPAD,), jnp.float32),
)


def _g1_body(x_ref, dis_ref, w_ref, out_ref):
    hw = jnp.dot(x_ref[...], w_ref[...], preferred_element_type=jnp.float32)
    out_ref[...] = dis_ref[...] * hw


_g1_call = pl.pallas_call(
    _g1_body,
    grid=(N // BM,),
    in_specs=[
        pl.BlockSpec((BM, D), lambda i: (i, 0)),
        pl.BlockSpec((BM, 1), lambda i: (i, 0)),
        pl.BlockSpec((D, D), lambda i: (0, 0)),
    ],
    out_specs=pl.BlockSpec((BM, D), lambda i: (i, 0)),
    out_shape=jax.ShapeDtypeStruct((N, D), jnp.float32),
)


def _mid_body(acc_ref, g_ref, dis_ref, b1_ref, w2_ref, out_ref):
    m = acc_ref[...] + g_ref[...]
    h = jnp.maximum(dis_ref[...] * m + b1_ref[...], 0.0)
    out_ref[...] = dis_ref[...] * jnp.dot(
        h, w2_ref[...], preferred_element_type=jnp.float32)


_mid_call = pl.pallas_call(
    _mid_body,
    grid=(N // BM,),
    in_specs=[
        pl.BlockSpec((BM, D), lambda i: (i, 0)),
        pl.BlockSpec((BM, D), lambda i: (i, 0)),
        pl.BlockSpec((BM, 1), lambda i: (i, 0)),
        pl.BlockSpec((1, D), lambda i: (0, 0)),
        pl.BlockSpec((D, D), lambda i: (0, 0)),
    ],
    out_specs=pl.BlockSpec((BM, D), lambda i: (i, 0)),
    out_shape=jax.ShapeDtypeStruct((N, D), jnp.float32),
)


def _elu(t):
    return jnp.where(t > 0, t, jnp.exp(jnp.minimum(t, 0.0)) - 1.0)


def _fin_body(acc_ref, g_ref, dis_ref, b2_ref, wm1_ref, bm1_ref,
              wm2_ref, bm2_ref, out_ref):
    m = acc_ref[...] + g_ref[...]
    h = _elu(dis_ref[...] * m + b2_ref[...])
    z = _elu(jnp.dot(h, wm1_ref[...], preferred_element_type=jnp.float32)
             + bm1_ref[...])
    logits = jnp.dot(z, wm2_ref[...], preferred_element_type=jnp.float32) \
        + bm2_ref[...]
    logits = logits - jnp.max(logits, axis=-1, keepdims=True)
    ez = jnp.exp(logits)
    out_ref[...] = ez / jnp.sum(ez, axis=-1, keepdims=True)


_fin_call = pl.pallas_call(
    _fin_body,
    grid=(N // BM,),
    in_specs=[
        pl.BlockSpec((BM, D), lambda i: (i, 0)),
        pl.BlockSpec((BM, D), lambda i: (i, 0)),
        pl.BlockSpec((BM, 1), lambda i: (i, 0)),
        pl.BlockSpec((1, D), lambda i: (0, 0)),
        pl.BlockSpec((D, D), lambda i: (0, 0)),
        pl.BlockSpec((1, D), lambda i: (0, 0)),
        pl.BlockSpec((D, K), lambda i: (0, 0)),
        pl.BlockSpec((1, K), lambda i: (0, 0)),
    ],
    out_specs=pl.BlockSpec((BM, K), lambda i: (i, 0)),
    out_shape=jax.ShapeDtypeStruct((N, K), jnp.float32),
)


def _pack_pairs(g):
    return jnp.stack([g[:, :DH].reshape(NP2, D), g[:, DH:].reshape(NP2, D)])


def _unpack_pairs(accp):
    return jnp.concatenate(
        [accp[0].reshape(NPAD, DH), accp[1].reshape(NPAD, DH)], axis=-1)


def kernel(x, edge_index, edge_attr, A, W1, b1, W2, b2, Wm1, bm1, Wm2, bm2):
    ei = edge_index.astype(jnp.int32)
    row = ei[0]
    col = ei[1]
    ew = edge_attr.astype(jnp.float32)
    pad = EPAD - E
    row_p = jnp.concatenate([row, jnp.zeros((pad,), jnp.int32)])
    col_p = jnp.concatenate([col, jnp.zeros((pad,), jnp.int32)])
    ew_p = jnp.concatenate([ew, jnp.zeros((pad,), jnp.float32)])
    rp_r = (row_p >> 1).reshape(NS, NCH, CH)
    cp_r = (col_p >> 1).reshape(NS, NCH, CH)
    par_r = ((row_p & 1) | ((col_p & 1) << 1)).reshape(NS, NCH, CH)
    ew_r = ew_p.reshape(NS, NCH, CH)

    degp = _sc_degree(col_p.reshape(NW, EPAD // NW),
                      ew_p.reshape(NW, EPAD // NW))
    dis2 = _dis_call(degp).reshape(NPAD, 1)
    g1 = _g1_call(x, dis2, W1)
    acc1 = _unpack_pairs(_sc_edge_pass(_pack_pairs(g1), rp_r, cp_r,
                                       par_r, ew_r))
    g2 = _mid_call(acc1, g1, dis2, b1.reshape(1, D), W2)
    acc2 = _unpack_pairs(_sc_edge_pass(_pack_pairs(g2), rp_r, cp_r,
                                       par_r, ew_r))
    S = _fin_call(acc2, g2, dis2, b2.reshape(1, D), Wm1,
                  bm1.reshape(1, D), Wm2, bm2.reshape(1, K))
    return (A, S)


# speedup vs baseline: 1.2622x; 1.2622x over previous
import functools

import jax
import jax.numpy as jnp
from jax import lax
from jax.experimental import pallas as pl
from jax.experimental.pallas import tpu as pltpu
from jax.experimental.pallas import tpu_sc as plsc

N = 10000
D = 128
DH = D // 2
K = 10
E = 320000

NC, NS, L = 2, 16, 16
NW = NC * NS
CH = 128
NCH = 160
NPH = 5
CPP = NCH // NPH
EPT = CH * NCH
EPAD = EPT * NS
NP2 = N // 2
NPAD2 = 5120
NPAD = 2 * NPAD2
RPS = NPAD2 // NS
BM = 1000

_sc_mesh = plsc.VectorSubcoreMesh(
    core_axis_name="c", subcore_axis_name="s", num_cores=NC, num_subcores=NS)
_sc_params = pltpu.CompilerParams(needs_layout_passes=False)


@functools.partial(
    pl.kernel,
    out_type=jax.ShapeDtypeStruct((NW, NPAD), jnp.float32),
    mesh=_sc_mesh,
    compiler_params=_sc_params,
    scratch_types=[
        pltpu.VMEM((EPAD // NW,), jnp.int32),
        pltpu.VMEM((EPAD // NW,), jnp.float32),
        pltpu.VMEM((NPAD,), jnp.float32),
    ],
)
def _sc_degree(col_hbm, ew_hbm, out_hbm, col_v, ew_v, deg_v):
    cid = lax.axis_index("c")
    sid = lax.axis_index("s")
    wid = sid * NC + cid
    pltpu.sync_copy(col_hbm.at[wid], col_v)
    pltpu.sync_copy(ew_hbm.at[wid], ew_v)
    zeros = jnp.zeros((L,), jnp.float32)

    def zero_body(k, _):
        deg_v[pl.ds(k * L, L)] = zeros
        return 0

    lax.fori_loop(0, NPAD // L, zero_body, 0)

    def edge_body(k, _):
        idx = col_v[pl.ds(k * L, L)]
        val = ew_v[pl.ds(k * L, L)]
        plsc.addupdate_scatter(deg_v, [idx], val)
        return 0

    lax.fori_loop(0, (EPAD // NW) // L, edge_body, 0)
    pltpu.sync_copy(deg_v, out_hbm.at[wid])


@functools.partial(
    pl.kernel,
    out_type=jax.ShapeDtypeStruct((NC, NPAD2, D), jnp.float32),
    mesh=_sc_mesh,
    compiler_params=_sc_params,
    scratch_types=[
        pltpu.VMEM((CPP, CH), jnp.int32),
        pltpu.VMEM((CPP, CH), jnp.int32),
        pltpu.VMEM((CPP, CH), jnp.int32),
        pltpu.VMEM((CPP, CH), jnp.float32),
        pltpu.VMEM((CH, D), jnp.float32),
        pltpu.VMEM((CH, D), jnp.float32),
        pltpu.VMEM_SHARED((NP2, D), jnp.float32),
        pltpu.VMEM_SHARED((NPAD2, D), jnp.float32),
        pltpu.SemaphoreType.DMA,
        pltpu.SemaphoreType.DMA,
    ],
)
def _sc_edge_pass(g_hbm, rp_hbm, cp_hbm, par_hbm, ew_hbm, out_hbm,
                  rp_v, cp_v, par_v, ew_v, buf0, buf1, g_sh, acc_sh,
                  sem0, sem1):
    cid = lax.axis_index("c")
    sid = lax.axis_index("s")

    @pl.when(sid < NS - 1)
    def _():
        off = pl.multiple_of(sid * 312, 8)
        pltpu.sync_copy(g_hbm.at[cid, pl.ds(off, 312)],
                        g_sh.at[pl.ds(off, 312)])

    @pl.when(sid == NS - 1)
    def _():
        pltpu.sync_copy(g_hbm.at[cid, pl.ds(4680, 320)],
                        g_sh.at[pl.ds(4680, 320)])

    zeros = jnp.zeros((L,), jnp.float32)

    def zrow(i, _):
        for v in range(D // L):
            buf0[i, pl.ds(v * L, L)] = zeros
        return 0

    lax.fori_loop(0, CH, zrow, 0)
    base = sid * RPS
    for j in range(RPS // 64):
        pltpu.sync_copy(buf0.at[pl.ds(0, 64)],
                        acc_sh.at[pl.ds(base + j * 64, 64)])
    plsc.subcore_barrier()

    bufs = (buf0, buf1)
    sems = (sem0, sem1)

    def scale(buf, j):
        def edge(b, _):
            w16 = ew_v[j, pl.ds(b * L, L)]
            p16 = par_v[j, pl.ds(b * L, L)]
            for lane in range(L):
                i = b * L + lane
                w = w16[lane]
                pv = p16[lane]
                soff = (pv & 1) * DH
                doff = (pv >> 1) * DH
                ooff = DH - doff
                vals = [buf[i, pl.ds(soff + v * L, L)]
                        for v in range(DH // L)]
                for v in range(DH // L):
                    buf[i, pl.ds(ooff + v * L, L)] = zeros
                for v in range(DH // L):
                    buf[i, pl.ds(doff + v * L, L)] = vals[v] * w
            return 0

        lax.fori_loop(0, CH // L, edge, 0)

    for ph in range(NPH):
        pltpu.sync_copy(rp_hbm.at[sid, pl.ds(ph * CPP, CPP)], rp_v)
        pltpu.sync_copy(cp_hbm.at[sid, pl.ds(ph * CPP, CPP)], cp_v)
        pltpu.sync_copy(par_hbm.at[sid, pl.ds(ph * CPP, CPP)], par_v)
        pltpu.sync_copy(ew_hbm.at[sid, pl.ds(ph * CPP, CPP)], ew_v)
        pltpu.async_copy(g_sh.at[rp_v.at[0]], bufs[0], sems[0])

        def pair(p, _):
            for q in range(2):
                j = 2 * p + q
                buf, nbuf = bufs[q], bufs[1 - q]
                sem, nsem = sems[q], sems[1 - q]
                pltpu.make_async_copy(g_sh.at[rp_v.at[j]], buf, sem).wait()

                @pl.when(j + 1 < CPP)
                def _():
                    pltpu.async_copy(g_sh.at[rp_v.at[j + 1]], nbuf, nsem)

                scale(buf, j)
                pltpu.sync_copy(buf, acc_sh.at[cp_v.at[j]], add=True)
            return 0

        lax.fori_loop(0, CPP // 2, pair, 0)

    plsc.subcore_barrier()
    for j in range(RPS // 64):
        sl = pl.ds(base + j * 64, 64)
        pltpu.sync_copy(acc_sh.at[sl], out_hbm.at[cid, sl])


def _dis_body(degp_ref, dis_ref):
    deg = jnp.sum(degp_ref[...], axis=0) + 1.0
    dis_ref[...] = jnp.where(deg > 0, lax.rsqrt(deg), 0.0)


_dis_call = pl.pallas_call(
    _dis_body,
    out_shape=jax.ShapeDtypeStruct((NPAD,), jnp.float32),
)


def _g1_body(x_ref, dis_ref, w_ref, out_ref):
    hw = jnp.dot(x_ref[...], w_ref[...], preferred_element_type=jnp.float32)
    out_ref[...] = dis_ref[...] * hw


_g1_call = pl.pallas_call(
    _g1_body,
    grid=(N // BM,),
    in_specs=[
        pl.BlockSpec((BM, D), lambda i: (i, 0)),
        pl.BlockSpec((BM, 1), lambda i: (i, 0)),
        pl.BlockSpec((D, D), lambda i: (0, 0)),
    ],
    out_specs=pl.BlockSpec((BM, D), lambda i: (i, 0)),
    out_shape=jax.ShapeDtypeStruct((N, D), jnp.float32),
)


def _mid_body(acc_ref, g_ref, dis_ref, b1_ref, w2_ref, out_ref):
    m = acc_ref[...] + g_ref[...]
    h = jnp.maximum(dis_ref[...] * m + b1_ref[...], 0.0)
    out_ref[...] = dis_ref[...] * jnp.dot(
        h, w2_ref[...], preferred_element_type=jnp.float32)


_mid_call = pl.pallas_call(
    _mid_body,
    grid=(N // BM,),
    in_specs=[
        pl.BlockSpec((BM, D), lambda i: (i, 0)),
        pl.BlockSpec((BM, D), lambda i: (i, 0)),
        pl.BlockSpec((BM, 1), lambda i: (i, 0)),
        pl.BlockSpec((1, D), lambda i: (0, 0)),
        pl.BlockSpec((D, D), lambda i: (0, 0)),
    ],
    out_specs=pl.BlockSpec((BM, D), lambda i: (i, 0)),
    out_shape=jax.ShapeDtypeStruct((N, D), jnp.float32),
)


def _elu(t):
    return jnp.where(t > 0, t, jnp.exp(jnp.minimum(t, 0.0)) - 1.0)


def _fin_body(acc_ref, g_ref, dis_ref, b2_ref, wm1_ref, bm1_ref,
              wm2_ref, bm2_ref, out_ref):
    m = acc_ref[...] + g_ref[...]
    h = _elu(dis_ref[...] * m + b2_ref[...])
    z = _elu(jnp.dot(h, wm1_ref[...], preferred_element_type=jnp.float32)
             + bm1_ref[...])
    logits = jnp.dot(z, wm2_ref[...], preferred_element_type=jnp.float32) \
        + bm2_ref[...]
    logits = logits - jnp.max(logits, axis=-1, keepdims=True)
    ez = jnp.exp(logits)
    out_ref[...] = ez / jnp.sum(ez, axis=-1, keepdims=True)


_fin_call = pl.pallas_call(
    _fin_body,
    grid=(N // BM,),
    in_specs=[
        pl.BlockSpec((BM, D), lambda i: (i, 0)),
        pl.BlockSpec((BM, D), lambda i: (i, 0)),
        pl.BlockSpec((BM, 1), lambda i: (i, 0)),
        pl.BlockSpec((1, D), lambda i: (0, 0)),
        pl.BlockSpec((D, D), lambda i: (0, 0)),
        pl.BlockSpec((1, D), lambda i: (0, 0)),
        pl.BlockSpec((D, K), lambda i: (0, 0)),
        pl.BlockSpec((1, K), lambda i: (0, 0)),
    ],
    out_specs=pl.BlockSpec((BM, K), lambda i: (i, 0)),
    out_shape=jax.ShapeDtypeStruct((N, K), jnp.float32),
)


def _pack_pairs(g):
    return jnp.stack([g[:, :DH].reshape(NP2, D), g[:, DH:].reshape(NP2, D)])


def _unpack_pairs(accp):
    return jnp.concatenate(
        [accp[0].reshape(NPAD, DH), accp[1].reshape(NPAD, DH)], axis=-1)


def kernel(x, edge_index, edge_attr, A, W1, b1, W2, b2, Wm1, bm1, Wm2, bm2):
    ei = edge_index.astype(jnp.int32)
    row = ei[0]
    col = ei[1]
    ew = edge_attr.astype(jnp.float32)
    pad = EPAD - E
    row_p = jnp.concatenate([row, jnp.zeros((pad,), jnp.int32)])
    col_p = jnp.concatenate([col, jnp.zeros((pad,), jnp.int32)])
    ew_p = jnp.concatenate([ew, jnp.zeros((pad,), jnp.float32)])
    rp_r = (row_p >> 1).reshape(NS, NCH, CH)
    cp_r = (col_p >> 1).reshape(NS, NCH, CH)
    par_r = ((row_p & 1) | ((col_p & 1) << 1)).reshape(NS, NCH, CH)
    ew_r = ew_p.reshape(NS, NCH, CH)

    degp = _sc_degree(col_p.reshape(NW, EPAD // NW),
                      ew_p.reshape(NW, EPAD // NW))
    dis2 = _dis_call(degp).reshape(NPAD, 1)
    g1 = _g1_call(x, dis2, W1)
    acc1 = _unpack_pairs(_sc_edge_pass(_pack_pairs(g1), rp_r, cp_r,
                                       par_r, ew_r))
    g2 = _mid_call(acc1, g1, dis2, b1.reshape(1, D), W2)
    acc2 = _unpack_pairs(_sc_edge_pass(_pack_pairs(g2), rp_r, cp_r,
                                       par_r, ew_r))
    S = _fin_call(acc2, g2, dis2, b2.reshape(1, D), Wm1,
                  bm1.reshape(1, D), Wm2, bm2.reshape(1, K))
    return (A, S)


# speedup vs baseline: 1.2678x; 1.0044x over previous
import functools

import jax
import jax.numpy as jnp
from jax import lax
from jax.experimental import pallas as pl
from jax.experimental.pallas import tpu as pltpu
from jax.experimental.pallas import tpu_sc as plsc

N = 10000
D = 128
DH = D // 2
K = 10
E = 320000

NC, NS, L = 2, 16, 16
NW = NC * NS
CH = 128
NCH = 160
NPH = 5
CPP = NCH // NPH
EPT = CH * NCH
EPAD = EPT * NS
NP2 = N // 2
NPAD2 = 5120
NPAD = 2 * NPAD2
RPS = NPAD2 // NS
BM = 1000

_sc_mesh = plsc.VectorSubcoreMesh(
    core_axis_name="c", subcore_axis_name="s", num_cores=NC, num_subcores=NS)
_sc_params = pltpu.CompilerParams(needs_layout_passes=False)


@functools.partial(
    pl.kernel,
    out_type=jax.ShapeDtypeStruct((NW, NPAD), jnp.float32),
    mesh=_sc_mesh,
    compiler_params=_sc_params,
    scratch_types=[
        pltpu.VMEM((EPAD // NW,), jnp.int32),
        pltpu.VMEM((EPAD // NW,), jnp.float32),
        pltpu.VMEM((NPAD,), jnp.float32),
    ],
)
def _sc_degree(col_hbm, ew_hbm, out_hbm, col_v, ew_v, deg_v):
    cid = lax.axis_index("c")
    sid = lax.axis_index("s")
    wid = sid * NC + cid
    pltpu.sync_copy(col_hbm.at[wid], col_v)
    pltpu.sync_copy(ew_hbm.at[wid], ew_v)
    zeros = jnp.zeros((L,), jnp.float32)

    def zero_body(k, _):
        deg_v[pl.ds(k * L, L)] = zeros
        return 0

    lax.fori_loop(0, NPAD // L, zero_body, 0)

    def edge_body(k, _):
        idx = col_v[pl.ds(k * L, L)]
        val = ew_v[pl.ds(k * L, L)]
        plsc.addupdate_scatter(deg_v, [idx], val)
        return 0

    lax.fori_loop(0, (EPAD // NW) // L, edge_body, 0)
    pltpu.sync_copy(deg_v, out_hbm.at[wid])


@functools.partial(
    pl.kernel,
    out_type=jax.ShapeDtypeStruct((NC, NPAD2, D), jnp.float32),
    mesh=_sc_mesh,
    compiler_params=_sc_params,
    scratch_types=[
        pltpu.VMEM((CPP, CH), jnp.int32),
        pltpu.VMEM((CPP, CH), jnp.int32),
        pltpu.VMEM((CPP, CH), jnp.int32),
        pltpu.VMEM((CPP, CH), jnp.float32),
        pltpu.VMEM((CH, D), jnp.float32),
        pltpu.VMEM((CH, D), jnp.float32),
        pltpu.VMEM_SHARED((NP2, D), jnp.float32),
        pltpu.VMEM_SHARED((NPAD2, D), jnp.float32),
        pltpu.SemaphoreType.DMA,
        pltpu.SemaphoreType.DMA,
        pltpu.SemaphoreType.DMA,
        pltpu.SemaphoreType.DMA,
    ],
)
def _sc_edge_pass(g_hbm, rp_hbm, cp_hbm, par_hbm, ew_hbm, out_hbm,
                  rp_v, cp_v, par_v, ew_v, buf0, buf1, g_sh, acc_sh,
                  sem0, sem1, ssem0, ssem1):
    cid = lax.axis_index("c")
    sid = lax.axis_index("s")

    @pl.when(sid < NS - 1)
    def _():
        off = pl.multiple_of(sid * 312, 8)
        pltpu.sync_copy(g_hbm.at[cid, pl.ds(off, 312)],
                        g_sh.at[pl.ds(off, 312)])

    @pl.when(sid == NS - 1)
    def _():
        pltpu.sync_copy(g_hbm.at[cid, pl.ds(4680, 320)],
                        g_sh.at[pl.ds(4680, 320)])

    zeros = jnp.zeros((L,), jnp.float32)

    def zrow(i, _):
        for v in range(D // L):
            buf0[i, pl.ds(v * L, L)] = zeros
        return 0

    lax.fori_loop(0, CH, zrow, 0)
    base = sid * RPS
    for j in range(RPS // 64):
        pltpu.sync_copy(buf0.at[pl.ds(0, 64)],
                        acc_sh.at[pl.ds(base + j * 64, 64)])
    plsc.subcore_barrier()

    bufs = (buf0, buf1)
    sems = (sem0, sem1)

    def scale(buf, j):
        def edge(b, _):
            w16 = ew_v[j, pl.ds(b * L, L)]
            p16 = par_v[j, pl.ds(b * L, L)]
            for lane in range(L):
                i = b * L + lane
                w = w16[lane]
                pv = p16[lane]
                soff = (pv & 1) * DH
                doff = (pv >> 1) * DH
                ooff = DH - doff
                vals = [buf[i, pl.ds(soff + v * L, L)]
                        for v in range(DH // L)]
                for v in range(DH // L):
                    buf[i, pl.ds(ooff + v * L, L)] = zeros
                for v in range(DH // L):
                    buf[i, pl.ds(doff + v * L, L)] = vals[v] * w
            return 0

        lax.fori_loop(0, CH // L, edge, 0)

    ssems = (ssem0, ssem1)

    def drain_scatter(b):
        pltpu.make_async_copy(g_hbm.at[cid, pl.ds(0, CH)],
                              bufs[b], ssems[b]).wait()

    for ph in range(NPH):
        if ph > 0:
            drain_scatter(0)
            drain_scatter(1)
        pltpu.sync_copy(rp_hbm.at[sid, pl.ds(ph * CPP, CPP)], rp_v)
        pltpu.sync_copy(cp_hbm.at[sid, pl.ds(ph * CPP, CPP)], cp_v)
        pltpu.sync_copy(par_hbm.at[sid, pl.ds(ph * CPP, CPP)], par_v)
        pltpu.sync_copy(ew_hbm.at[sid, pl.ds(ph * CPP, CPP)], ew_v)
        pltpu.async_copy(g_sh.at[rp_v.at[0]], bufs[0], sems[0])

        def pair(p, _):
            for q in range(2):
                j = 2 * p + q
                buf, nbuf = bufs[q], bufs[1 - q]
                sem, nsem = sems[q], sems[1 - q]
                pltpu.make_async_copy(g_sh.at[rp_v.at[j]], buf, sem).wait()

                @pl.when(jnp.logical_and(j + 1 < CPP, j >= 1))
                def _():
                    pltpu.make_async_copy(
                        g_hbm.at[cid, pl.ds(0, CH)], nbuf,
                        ssems[1 - q]).wait()

                @pl.when(j + 1 < CPP)
                def _():
                    pltpu.async_copy(g_sh.at[rp_v.at[j + 1]], nbuf, nsem)

                scale(buf, j)
                pltpu.async_copy(buf, acc_sh.at[cp_v.at[j]], ssems[q],
                                 add=True)
            return 0

        lax.fori_loop(0, CPP // 2, pair, 0)

    drain_scatter(0)
    drain_scatter(1)
    plsc.subcore_barrier()
    for j in range(RPS // 64):
        sl = pl.ds(base + j * 64, 64)
        pltpu.sync_copy(acc_sh.at[sl], out_hbm.at[cid, sl])


def _dis_body(degp_ref, dis_ref):
    deg = jnp.sum(degp_ref[...], axis=0) + 1.0
    dis_ref[...] = jnp.where(deg > 0, lax.rsqrt(deg), 0.0)


_dis_call = pl.pallas_call(
    _dis_body,
    out_shape=jax.ShapeDtypeStruct((NPAD,), jnp.float32),
)


def _g1_body(x_ref, dis_ref, w_ref, out_ref):
    hw = jnp.dot(x_ref[...], w_ref[...], preferred_element_type=jnp.float32)
    out_ref[...] = dis_ref[...] * hw


_g1_call = pl.pallas_call(
    _g1_body,
    grid=(N // BM,),
    in_specs=[
        pl.BlockSpec((BM, D), lambda i: (i, 0)),
        pl.BlockSpec((BM, 1), lambda i: (i, 0)),
        pl.BlockSpec((D, D), lambda i: (0, 0)),
    ],
    out_specs=pl.BlockSpec((BM, D), lambda i: (i, 0)),
    out_shape=jax.ShapeDtypeStruct((N, D), jnp.float32),
)


def _mid_body(acc_ref, g_ref, dis_ref, b1_ref, w2_ref, out_ref):
    m = acc_ref[...] + g_ref[...]
    h = jnp.maximum(dis_ref[...] * m + b1_ref[...], 0.0)
    out_ref[...] = dis_ref[...] * jnp.dot(
        h, w2_ref[...], preferred_element_type=jnp.float32)


_mid_call = pl.pallas_call(
    _mid_body,
    grid=(N // BM,),
    in_specs=[
        pl.BlockSpec((BM, D), lambda i: (i, 0)),
        pl.BlockSpec((BM, D), lambda i: (i, 0)),
        pl.BlockSpec((BM, 1), lambda i: (i, 0)),
        pl.BlockSpec((1, D), lambda i: (0, 0)),
        pl.BlockSpec((D, D), lambda i: (0, 0)),
    ],
    out_specs=pl.BlockSpec((BM, D), lambda i: (i, 0)),
    out_shape=jax.ShapeDtypeStruct((N, D), jnp.float32),
)


def _elu(t):
    return jnp.where(t > 0, t, jnp.exp(jnp.minimum(t, 0.0)) - 1.0)


def _fin_body(acc_ref, g_ref, dis_ref, b2_ref, wm1_ref, bm1_ref,
              wm2_ref, bm2_ref, out_ref):
    m = acc_ref[...] + g_ref[...]
    h = _elu(dis_ref[...] * m + b2_ref[...])
    z = _elu(jnp.dot(h, wm1_ref[...], preferred_element_type=jnp.float32)
             + bm1_ref[...])
    logits = jnp.dot(z, wm2_ref[...], preferred_element_type=jnp.float32) \
        + bm2_ref[...]
    logits = logits - jnp.max(logits, axis=-1, keepdims=True)
    ez = jnp.exp(logits)
    out_ref[...] = ez / jnp.sum(ez, axis=-1, keepdims=True)


_fin_call = pl.pallas_call(
    _fin_body,
    grid=(N // BM,),
    in_specs=[
        pl.BlockSpec((BM, D), lambda i: (i, 0)),
        pl.BlockSpec((BM, D), lambda i: (i, 0)),
        pl.BlockSpec((BM, 1), lambda i: (i, 0)),
        pl.BlockSpec((1, D), lambda i: (0, 0)),
        pl.BlockSpec((D, D), lambda i: (0, 0)),
        pl.BlockSpec((1, D), lambda i: (0, 0)),
        pl.BlockSpec((D, K), lambda i: (0, 0)),
        pl.BlockSpec((1, K), lambda i: (0, 0)),
    ],
    out_specs=pl.BlockSpec((BM, K), lambda i: (i, 0)),
    out_shape=jax.ShapeDtypeStruct((N, K), jnp.float32),
)


def _pack_pairs(g):
    return jnp.stack([g[:, :DH].reshape(NP2, D), g[:, DH:].reshape(NP2, D)])


def _unpack_pairs(accp):
    return jnp.concatenate(
        [accp[0].reshape(NPAD, DH), accp[1].reshape(NPAD, DH)], axis=-1)


def kernel(x, edge_index, edge_attr, A, W1, b1, W2, b2, Wm1, bm1, Wm2, bm2):
    ei = edge_index.astype(jnp.int32)
    row = ei[0]
    col = ei[1]
    ew = edge_attr.astype(jnp.float32)
    pad = EPAD - E
    row_p = jnp.concatenate([row, jnp.zeros((pad,), jnp.int32)])
    col_p = jnp.concatenate([col, jnp.zeros((pad,), jnp.int32)])
    ew_p = jnp.concatenate([ew, jnp.zeros((pad,), jnp.float32)])
    rp_r = (row_p >> 1).reshape(NS, NCH, CH)
    cp_r = (col_p >> 1).reshape(NS, NCH, CH)
    par_r = ((row_p & 1) | ((col_p & 1) << 1)).reshape(NS, NCH, CH)
    ew_r = ew_p.reshape(NS, NCH, CH)

    degp = _sc_degree(col_p.reshape(NW, EPAD // NW),
                      ew_p.reshape(NW, EPAD // NW))
    dis2 = _dis_call(degp).reshape(NPAD, 1)
    g1 = _g1_call(x, dis2, W1)
    acc1 = _unpack_pairs(_sc_edge_pass(_pack_pairs(g1), rp_r, cp_r,
                                       par_r, ew_r))
    g2 = _mid_call(acc1, g1, dis2, b1.reshape(1, D), W2)
    acc2 = _unpack_pairs(_sc_edge_pass(_pack_pairs(g2), rp_r, cp_r,
                                       par_r, ew_r))
    S = _fin_call(acc2, g2, dis2, b2.reshape(1, D), Wm1,
                  bm1.reshape(1, D), Wm2, bm2.reshape(1, K))
    return (A, S)
